# Initial kernel scaffold; baseline (speedup 1.0000x reference)
import functools

import jax
import jax.numpy as jnp
from jax import lax
from jax.experimental import pallas as pl
from jax.experimental.pallas import tpu as pltpu
from jax.experimental.pallas import tpu_sc as plsc

_N = 20000
_E = 320000
_F = 128
_NHEADS = 4
_NH = 16
_NC = 2
_NS = 16
_NW = _NC * _NS
_HALF = _N // _NC
_DUMP = _HALF
_AROWS = _HALF + _NS
_ZST = _AROWS // _NS
_WST = _HALF // _NS
_CH = 80

_mesh = plsc.VectorSubcoreMesh(core_axis_name="c", subcore_axis_name="s")


def _mm_body(x_ref, w_ref, b_ref, o_ref):
  o_ref[...] = (
      jnp.dot(x_ref[...], w_ref[...], preferred_element_type=jnp.float32)
      + b_ref[...])


def _mm(x, w, b, bm=2000):
  m, k = x.shape
  n = w.shape[1]
  return pl.pallas_call(
      _mm_body,
      grid=(m // bm,),
      in_specs=[
          pl.BlockSpec((bm, k), lambda i: (i, 0)),
          pl.BlockSpec((k, n), lambda i: (0, 0)),
          pl.BlockSpec((1, n), lambda i: (0, 0)),
      ],
      out_specs=pl.BlockSpec((bm, n), lambda i: (i, 0)),
      out_shape=jax.ShapeDtypeStruct((m, n), jnp.float32),
  )(x, w, b.reshape(1, n))


def _fsfd_body(hs_ref, hd_ref, ws_ref, wd_ref, fs_ref, fd_ref):
  fs_ref[...] = jnp.dot(hs_ref[...], ws_ref[...],
                        preferred_element_type=jnp.float32)
  fd_ref[...] = jnp.dot(hd_ref[...], wd_ref[...],
                        preferred_element_type=jnp.float32)


def _fsfd(h_src, h_dst, ws, wd, bm=2000):
  return pl.pallas_call(
      _fsfd_body,
      grid=(_N // bm,),
      in_specs=[
          pl.BlockSpec((bm, _F), lambda i: (i, 0)),
          pl.BlockSpec((bm, _F), lambda i: (i, 0)),
          pl.BlockSpec((_F, _F), lambda i: (0, 0)),
          pl.BlockSpec((_F, _F), lambda i: (0, 0)),
      ],
      out_specs=[
          pl.BlockSpec((bm, _F), lambda i: (i, 0)),
          pl.BlockSpec((bm, _F), lambda i: (i, 0)),
      ],
      out_shape=[
          jax.ShapeDtypeStruct((_N, _F), jnp.float32),
          jax.ShapeDtypeStruct((_N, _F), jnp.float32),
      ],
  )(h_src, h_dst, ws, wd)


def _edge_body(fs_ref, fd_ref, abd_ref, bind_ref, w_ref, x_ref):
  t = fs_ref[...] + fd_ref[...]
  t = jnp.maximum(t, 0.2 * t)
  l16 = jnp.dot(t, abd_ref[...], preferred_element_type=jnp.float32)
  col = lax.broadcasted_iota(jnp.int32, l16.shape, 1)
  ex = jnp.where(col < _NHEADS, jnp.exp(l16), 0.0)
  x_ref[...] = ex
  w_ref[...] = fs_ref[...] * jnp.dot(ex, bind_ref[...],
                                     preferred_element_type=jnp.float32)


def _edge(FS, FD, abd, bind, bm=4000):
  return pl.pallas_call(
      _edge_body,
      grid=(_E // bm,),
      in_specs=[
          pl.BlockSpec((bm, _F), lambda i: (i, 0)),
          pl.BlockSpec((bm, _F), lambda i: (i, 0)),
          pl.BlockSpec((_F, _NH), lambda i: (0, 0)),
          pl.BlockSpec((_NH, _F), lambda i: (0, 0)),
      ],
      out_specs=[
          pl.BlockSpec((bm, _F), lambda i: (i, 0)),
          pl.BlockSpec((bm, _NH), lambda i: (i, 0)),
      ],
      out_shape=[
          jax.ShapeDtypeStruct((_E, _F), jnp.float32),
          jax.ShapeDtypeStruct((_E, _NH), jnp.float32),
      ],
  )(FS, FD, abd, bind)


def _fin_body(aw_ref, ax_ref, bind_ref, b_ref, o_ref):
  s = jnp.dot(ax_ref[...], bind_ref[...], preferred_element_type=jnp.float32)
  o_ref[...] = aw_ref[...] / (s + 1e-9) + b_ref[...]


def _finalize(aw, ax, bind, b, bm=2000):
  return pl.pallas_call(
      _fin_body,
      grid=(_N // bm,),
      in_specs=[
          pl.BlockSpec((bm, _F), lambda i: (i, 0)),
          pl.BlockSpec((bm, _NH), lambda i: (i, 0)),
          pl.BlockSpec((_NH, _F), lambda i: (0, 0)),
          pl.BlockSpec((1, _F), lambda i: (0, 0)),
      ],
      out_specs=pl.BlockSpec((bm, _F), lambda i: (i, 0)),
      out_shape=jax.ShapeDtypeStruct((_N, _F), jnp.float32),
  )(aw, ax, bind, b.reshape(1, _F))


def _clf_body(h_ref, w1_ref, b1_ref, w2_ref, b2_ref, o_ref):
  x = jnp.dot(h_ref[...], w1_ref[...], preferred_element_type=jnp.float32)
  x = jnp.maximum(x + b1_ref[...], 0.0)
  o_ref[...] = (
      jnp.dot(x, w2_ref[...], preferred_element_type=jnp.float32)
      + b2_ref[...])


def _clf(h, w1, b1, w2, b2, bm=2000):
  n1 = w1.shape[1]
  n2 = w2.shape[1]
  return pl.pallas_call(
      _clf_body,
      grid=(_N // bm,),
      in_specs=[
          pl.BlockSpec((bm, _F), lambda i: (i, 0)),
          pl.BlockSpec((_F, n1), lambda i: (0, 0)),
          pl.BlockSpec((1, n1), lambda i: (0, 0)),
          pl.BlockSpec((n1, n2), lambda i: (0, 0)),
          pl.BlockSpec((1, n2), lambda i: (0, 0)),
      ],
      out_specs=pl.BlockSpec((bm, n2), lambda i: (i, 0)),
      out_shape=jax.ShapeDtypeStruct((_N, n2), jnp.float32),
  )(h, w1, b1.reshape(1, n1), w2, b2.reshape(1, n2))


def _gather_impl(fs_hbm, fd_hbm, src_hbm, dst_hbm, FS_hbm, FD_hbm,
                 sidx, didx, srows, drows, sem1, sem2):
  c = lax.axis_index("c")
  s = lax.axis_index("s")
  wid = s * _NC + c
  epw = _E // _NW
  base = wid * epw

  def body(j, carry):
    off = base + j * _CH
    pltpu.sync_copy(src_hbm.at[pl.ds(off, _CH)], sidx)
    pltpu.sync_copy(dst_hbm.at[pl.ds(off, _CH)], didx)
    cp1 = pltpu.async_copy(fs_hbm.at[sidx], srows, sem1)
    cp2 = pltpu.async_copy(fd_hbm.at[didx], drows, sem2)
    cp1.wait()
    cp2.wait()
    pltpu.sync_copy(srows, FS_hbm.at[pl.ds(off, _CH)])
    pltpu.sync_copy(drows, FD_hbm.at[pl.ds(off, _CH)])
    return carry

  lax.fori_loop(0, epw // _CH, body, 0)


@functools.partial(
    pl.kernel,
    out_type=[
        jax.ShapeDtypeStruct((_E, _F), jnp.float32),
        jax.ShapeDtypeStruct((_E, _F), jnp.float32),
    ],
    mesh=_mesh,
    scratch_types=[
        pltpu.VMEM((_CH,), jnp.int32),
        pltpu.VMEM((_CH,), jnp.int32),
        pltpu.VMEM((_CH, _F), jnp.float32),
        pltpu.VMEM((_CH, _F), jnp.float32),
        pltpu.SemaphoreType.DMA,
        pltpu.SemaphoreType.DMA,
    ],
)
def _gather2(fs_hbm, fd_hbm, src_hbm, dst_hbm, FS_hbm, FD_hbm,
             sidx, didx, srows, drows, sem1, sem2):
  _gather_impl(fs_hbm, fd_hbm, src_hbm, dst_hbm, FS_hbm, FD_hbm,
               sidx, didx, srows, drows, sem1, sem2)


def _scatter_impl(W_hbm, X_hbm, dst_hbm, zw_hbm, zx_hbm, outw_hbm, outx_hbm,
                  accw, accx, idx, lidx, wrows, xrows):
  c = lax.axis_index("c")
  s = lax.axis_index("s")
  pltpu.sync_copy(zw_hbm.at[pl.ds(s * _ZST, _ZST)],
                  accw.at[pl.ds(s * _ZST, _ZST)])
  pltpu.sync_copy(zx_hbm.at[pl.ds(s * _ZST, _ZST)],
                  accx.at[pl.ds(s * _ZST, _ZST)])
  plsc.subcore_barrier()

  lo = (c * _HALF).astype(jnp.int32)
  eps = _E // _NS

  def body(j, carry):
    off = s * eps + j * _CH
    pltpu.sync_copy(dst_hbm.at[pl.ds(off, _CH)], idx)
    pltpu.sync_copy(W_hbm.at[pl.ds(off, _CH)], wrows)
    pltpu.sync_copy(X_hbm.at[pl.ds(off, _CH)], xrows)
    for l in range(_CH // 16):
      v = idx[pl.ds(l * 16, 16)] - lo
      ok = (v >= 0) & (v < _HALF)
      lidx[pl.ds(l * 16, 16)] = jnp.where(ok, v, _DUMP)
    pltpu.sync_copy(wrows, accw.at[lidx], add=True)
    pltpu.sync_copy(xrows, accx.at[lidx], add=True)
    return carry

  lax.fori_loop(0, eps // _CH, body, 0)
  plsc.subcore_barrier()

  r = s * _WST
  pltpu.sync_copy(accw.at[pl.ds(r, _WST)],
                  outw_hbm.at[pl.ds(c * _HALF + r, _WST)])
  pltpu.sync_copy(accx.at[pl.ds(r, _WST)],
                  outx_hbm.at[pl.ds(c * _HALF + r, _WST)])


@functools.partial(
    pl.kernel,
    out_type=[
        jax.ShapeDtypeStruct((_N, _F), jnp.float32),
        jax.ShapeDtypeStruct((_N, _NH), jnp.float32),
    ],
    mesh=_mesh,
    scratch_types=[
        pltpu.VMEM_SHARED((_AROWS, _F), jnp.float32),
        pltpu.VMEM_SHARED((_AROWS, _NH), jnp.float32),
        pltpu.VMEM((_CH,), jnp.int32),
        pltpu.VMEM((_CH,), jnp.int32),
        pltpu.VMEM((_CH, _F), jnp.float32),
        pltpu.VMEM((_CH, _NH), jnp.float32),
    ],
)
def _scatter2(W_hbm, X_hbm, dst_hbm, zw_hbm, zx_hbm, outw_hbm, outx_hbm,
              accw, accx, idx, lidx, wrows, xrows):
  _scatter_impl(W_hbm, X_hbm, dst_hbm, zw_hbm, zx_hbm, outw_hbm, outx_hbm,
                accw, accx, idx, lidx, wrows, xrows)


def _abd(attn):
  a = jnp.zeros((_F, _NH), jnp.float32)
  for h in range(_NHEADS):
    a = a.at[h * 32:(h + 1) * 32, h].set(attn[h])
  return a


def _bind():
  b = jnp.zeros((_NH, _F), jnp.float32)
  for h in range(_NHEADS):
    b = b.at[h, h * 32:(h + 1) * 32].set(1.0)
  return b


def kernel(h_document, h_word, edge_index_w2d, edge_index_d2w, params):
  pr = params['proj']
  hd0 = _mm(h_document, pr['document']['W'], pr['document']['b'])
  hw0 = _mm(h_word, pr['word']['W'], pr['word']['b'])

  zw = jnp.zeros((_AROWS, _F), jnp.float32)
  zx = jnp.zeros((_AROWS, _NH), jnp.float32)
  bind = _bind()

  def gatv2(h_src, h_dst, src, dst, p):
    fs, fd = _fsfd(h_src, h_dst, p['Ws'], p['Wd'])
    FS, FD = _gather2(fs, fd, src, dst)
    Wr, Xr = _edge(FS, FD, _abd(p['attn']), bind)
    aw, ax = _scatter2(Wr, Xr, dst, zw, zx)
    return _finalize(aw, ax, bind, p['b'])

  s_w2d = edge_index_w2d[0].astype(jnp.int32)
  d_w2d = edge_index_w2d[1].astype(jnp.int32)
  s_d2w = edge_index_d2w[0].astype(jnp.int32)
  d_d2w = edge_index_d2w[1].astype(jnp.int32)

  L = params['layers']
  hd1 = gatv2(hw0, hd0, s_w2d, d_w2d, L[0]['w2d'])
  hw1 = gatv2(hd0, hw0, s_d2w, d_d2w, L[0]['d2w'])
  hd2 = gatv2(hw1, hd1, s_w2d, d_w2d, L[1]['w2d'])

  c = params['clf']
  return _clf(hd2, c['W1'], c['b1'], c['W2'], c['b2'])


# speedup vs baseline: 24.8922x; 24.8922x over previous
import functools

import jax
import jax.numpy as jnp
from jax import lax
from jax.experimental import pallas as pl
from jax.experimental.pallas import tpu as pltpu
from jax.experimental.pallas import tpu_sc as plsc

_N = 20000
_E = 320000
_F = 128
_NHEADS = 4
_NC = 2
_NS = 16
_NW = _NC * _NS
_HALF = _N // _NC
_DUMP = _HALF
_AROWS = _HALF + 8
_ST = 624
_ZTAIL = _AROWS - _NS * _ST
_WTAIL = _HALF - _NS * _ST
_CH = 80

@functools.cache
def _sc_mesh():
  return plsc.VectorSubcoreMesh(core_axis_name="c", subcore_axis_name="s",
                                num_cores=_NC, num_subcores=_NS)


def _mm_body(x_ref, w_ref, b_ref, o_ref):
  o_ref[...] = (
      jnp.dot(x_ref[...], w_ref[...], preferred_element_type=jnp.float32)
      + b_ref[...])


def _mm(x, w, b, bm=2000):
  m, k = x.shape
  n = w.shape[1]
  return pl.pallas_call(
      _mm_body,
      grid=(m // bm,),
      in_specs=[
          pl.BlockSpec((bm, k), lambda i: (i, 0)),
          pl.BlockSpec((k, n), lambda i: (0, 0)),
          pl.BlockSpec((1, n), lambda i: (0, 0)),
      ],
      out_specs=pl.BlockSpec((bm, n), lambda i: (i, 0)),
      out_shape=jax.ShapeDtypeStruct((m, n), jnp.float32),
  )(x, w, b.reshape(1, n))


def _fsfd_body(hs_ref, hd_ref, ws_ref, wd_ref, fs_ref, fd_ref):
  fs_ref[...] = jnp.dot(hs_ref[...], ws_ref[...],
                        preferred_element_type=jnp.float32)
  fd_ref[...] = jnp.dot(hd_ref[...], wd_ref[...],
                        preferred_element_type=jnp.float32)


def _fsfd(h_src, h_dst, ws, wd, bm=2000):
  return pl.pallas_call(
      _fsfd_body,
      grid=(_N // bm,),
      in_specs=[
          pl.BlockSpec((bm, _F), lambda i: (i, 0)),
          pl.BlockSpec((bm, _F), lambda i: (i, 0)),
          pl.BlockSpec((_F, _F), lambda i: (0, 0)),
          pl.BlockSpec((_F, _F), lambda i: (0, 0)),
      ],
      out_specs=[
          pl.BlockSpec((bm, _F), lambda i: (i, 0)),
          pl.BlockSpec((bm, _F), lambda i: (i, 0)),
      ],
      out_shape=[
          jax.ShapeDtypeStruct((_N, _F), jnp.float32),
          jax.ShapeDtypeStruct((_N, _F), jnp.float32),
      ],
  )(h_src, h_dst, ws, wd)


def _edge_body(fs_ref, fd_ref, abd_ref, bind_ref, w_ref, x_ref):
  t = fs_ref[...] + fd_ref[...]
  t = jnp.maximum(t, 0.2 * t)
  l = jnp.dot(t, abd_ref[...], preferred_element_type=jnp.float32)
  col = lax.broadcasted_iota(jnp.int32, l.shape, 1)
  ex = jnp.where(col < _NHEADS, jnp.exp(l), 0.0)
  x_ref[...] = ex
  w_ref[...] = fs_ref[...] * jnp.dot(ex, bind_ref[...],
                                     preferred_element_type=jnp.float32)


def _edge(FS, FD, abd, bind, bm=4000):
  return pl.pallas_call(
      _edge_body,
      grid=(_E // bm,),
      in_specs=[
          pl.BlockSpec((bm, _F), lambda i: (i, 0)),
          pl.BlockSpec((bm, _F), lambda i: (i, 0)),
          pl.BlockSpec((_F, _F), lambda i: (0, 0)),
          pl.BlockSpec((_F, _F), lambda i: (0, 0)),
      ],
      out_specs=[
          pl.BlockSpec((bm, _F), lambda i: (i, 0)),
          pl.BlockSpec((bm, _F), lambda i: (i, 0)),
      ],
      out_shape=[
          jax.ShapeDtypeStruct((_E, _F), jnp.float32),
          jax.ShapeDtypeStruct((_E, _F), jnp.float32),
      ],
  )(FS, FD, abd, bind)


def _fin_body(aw_ref, ax_ref, bind_ref, b_ref, o_ref):
  s = jnp.dot(ax_ref[...], bind_ref[...], preferred_element_type=jnp.float32)
  o_ref[...] = aw_ref[...] / (s + 1e-9) + b_ref[...]


def _finalize(aw, ax, bind, b, bm=2000):
  return pl.pallas_call(
      _fin_body,
      grid=(_N // bm,),
      in_specs=[
          pl.BlockSpec((bm, _F), lambda i: (i, 0)),
          pl.BlockSpec((bm, _F), lambda i: (i, 0)),
          pl.BlockSpec((_F, _F), lambda i: (0, 0)),
          pl.BlockSpec((1, _F), lambda i: (0, 0)),
      ],
      out_specs=pl.BlockSpec((bm, _F), lambda i: (i, 0)),
      out_shape=jax.ShapeDtypeStruct((_N, _F), jnp.float32),
  )(aw, ax, bind, b.reshape(1, _F))


def _clf_body(h_ref, w1_ref, b1_ref, w2_ref, b2_ref, o_ref):
  x = jnp.dot(h_ref[...], w1_ref[...], preferred_element_type=jnp.float32)
  x = jnp.maximum(x + b1_ref[...], 0.0)
  o_ref[...] = (
      jnp.dot(x, w2_ref[...], preferred_element_type=jnp.float32)
      + b2_ref[...])


def _clf(h, w1, b1, w2, b2, bm=2000):
  n1 = w1.shape[1]
  n2 = w2.shape[1]
  return pl.pallas_call(
      _clf_body,
      grid=(_N // bm,),
      in_specs=[
          pl.BlockSpec((bm, _F), lambda i: (i, 0)),
          pl.BlockSpec((_F, n1), lambda i: (0, 0)),
          pl.BlockSpec((1, n1), lambda i: (0, 0)),
          pl.BlockSpec((n1, n2), lambda i: (0, 0)),
          pl.BlockSpec((1, n2), lambda i: (0, 0)),
      ],
      out_specs=pl.BlockSpec((bm, n2), lambda i: (i, 0)),
      out_shape=jax.ShapeDtypeStruct((_N, n2), jnp.float32),
  )(h, w1, b1.reshape(1, n1), w2, b2.reshape(1, n2))


def _gather_impl(fs_hbm, fd_hbm, src_hbm, dst_hbm, FS_hbm, FD_hbm,
                 sidx, didx, srows, drows, sem1, sem2):
  c = lax.axis_index("c")
  s = lax.axis_index("s")
  wid = s * _NC + c
  epw = _E // _NW
  base = wid * epw

  def body(j, carry):
    off = base + j * _CH
    pltpu.sync_copy(src_hbm.at[pl.ds(off, _CH)], sidx)
    pltpu.sync_copy(dst_hbm.at[pl.ds(off, _CH)], didx)
    cp1 = pltpu.async_copy(fs_hbm.at[sidx], srows, sem1)
    cp2 = pltpu.async_copy(fd_hbm.at[didx], drows, sem2)
    cp1.wait()
    cp2.wait()
    pltpu.sync_copy(srows, FS_hbm.at[pl.ds(off, _CH)])
    pltpu.sync_copy(drows, FD_hbm.at[pl.ds(off, _CH)])
    return carry

  lax.fori_loop(0, epw // _CH, body, 0)


@functools.cache
def _gather2_kernel():
  return pl.kernel(
      _gather_impl,
      out_type=[
          jax.ShapeDtypeStruct((_E, _F), jnp.float32),
          jax.ShapeDtypeStruct((_E, _F), jnp.float32),
      ],
      mesh=_sc_mesh(),
      scratch_types=[
          pltpu.VMEM((_CH,), jnp.int32),
          pltpu.VMEM((_CH,), jnp.int32),
          pltpu.VMEM((_CH, _F), jnp.float32),
          pltpu.VMEM((_CH, _F), jnp.float32),
          pltpu.SemaphoreType.DMA,
          pltpu.SemaphoreType.DMA,
      ],
  )


def _gather2(fs, fd, src, dst):
  return _gather2_kernel()(fs, fd, src, dst)


def _scatter_impl(W_hbm, dst_hbm, z_hbm, out_hbm, acc, idx, lidx, rows):
  c = lax.axis_index("c")
  s = lax.axis_index("s")
  pltpu.sync_copy(z_hbm.at[pl.ds(s * _ST, _ST)], acc.at[pl.ds(s * _ST, _ST)])

  @pl.when(s == 0)
  def _zero_tail():
    pltpu.sync_copy(z_hbm.at[pl.ds(_NS * _ST, _ZTAIL)],
                    acc.at[pl.ds(_NS * _ST, _ZTAIL)])

  plsc.subcore_barrier()

  lo = (c * _HALF).astype(jnp.int32)
  eps = _E // _NS

  def body(j, carry):
    off = s * eps + j * _CH
    pltpu.sync_copy(dst_hbm.at[pl.ds(off, _CH)], idx)
    pltpu.sync_copy(W_hbm.at[pl.ds(off, _CH)], rows)
    for l in range(_CH // 16):
      v = idx[pl.ds(l * 16, 16)] - lo
      ok = (v >= 0) & (v < _HALF)
      lidx[pl.ds(l * 16, 16)] = jnp.where(ok, v, _DUMP)
    pltpu.sync_copy(rows, acc.at[lidx], add=True)
    return carry

  lax.fori_loop(0, eps // _CH, body, 0)
  plsc.subcore_barrier()

  r = s * _ST
  pltpu.sync_copy(acc.at[pl.ds(r, _ST)],
                  out_hbm.at[pl.ds(c * _HALF + r, _ST)])

  @pl.when(s == 0)
  def _write_tail():
    pltpu.sync_copy(acc.at[pl.ds(_NS * _ST, _WTAIL)],
                    out_hbm.at[pl.ds(c * _HALF + _NS * _ST, _WTAIL)])


@functools.cache
def _scatter_kernel():
  return pl.kernel(
      _scatter_impl,
      out_type=jax.ShapeDtypeStruct((_N, _F), jnp.float32),
      mesh=_sc_mesh(),
      scratch_types=[
          pltpu.VMEM_SHARED((_AROWS, _F), jnp.float32),
          pltpu.VMEM((_CH,), jnp.int32),
          pltpu.VMEM((_CH,), jnp.int32),
          pltpu.VMEM((_CH, _F), jnp.float32),
      ],
  )


def _scatter128(W, dst, z):
  return _scatter_kernel()(W, dst, z)


def _abd(attn):
  a = jnp.zeros((_F, _F), jnp.float32)
  for h in range(_NHEADS):
    a = a.at[h * 32:(h + 1) * 32, h].set(attn[h])
  return a


def _bind():
  b = jnp.zeros((_F, _F), jnp.float32)
  for h in range(_NHEADS):
    b = b.at[h, h * 32:(h + 1) * 32].set(1.0)
  return b


def kernel(h_document, h_word, edge_index_w2d, edge_index_d2w, params):
  pr = params['proj']
  hd0 = _mm(h_document, pr['document']['W'], pr['document']['b'])
  hw0 = _mm(h_word, pr['word']['W'], pr['word']['b'])

  z = jnp.zeros((_AROWS, _F), jnp.float32)
  bind = _bind()

  def gatv2(h_src, h_dst, src, dst, p):
    fs, fd = _fsfd(h_src, h_dst, p['Ws'], p['Wd'])
    FS, FD = _gather2(fs, fd, src, dst)
    Wr, Xr = _edge(FS, FD, _abd(p['attn']), bind)
    aw = _scatter128(Wr, dst, z)
    ax = _scatter128(Xr, dst, z)
    return _finalize(aw, ax, bind, p['b'])

  s_w2d = edge_index_w2d[0].astype(jnp.int32)
  d_w2d = edge_index_w2d[1].astype(jnp.int32)
  s_d2w = edge_index_d2w[0].astype(jnp.int32)
  d_d2w = edge_index_d2w[1].astype(jnp.int32)

  L = params['layers']
  hd1 = gatv2(hw0, hd0, s_w2d, d_w2d, L[0]['w2d'])
  hw1 = gatv2(hd0, hw0, s_d2w, d_d2w, L[0]['d2w'])
  hd2 = gatv2(hw1, hd1, s_w2d, d_w2d, L[1]['w2d'])

  c = params['clf']
  return _clf(hd2, c['W1'], c['b1'], c['W2'], c['b2'])


# speedup vs baseline: 40.8623x; 1.6416x over previous
import functools

import jax
import jax.numpy as jnp
from jax import lax
from jax.experimental import pallas as pl
from jax.experimental.pallas import tpu as pltpu
from jax.experimental.pallas import tpu_sc as plsc

_N = 20000
_E = 320000
_F = 128
_NHEADS = 4
_NC = 2
_NS = 16
_NW = _NC * _NS
_HALF = _N // _NC
_DUMP = _HALF
_AROWS = _HALF + 8
_ST = 624
_ZTAIL = _AROWS - _NS * _ST
_WTAIL = _HALF - _NS * _ST
_CH = 128
_GFULL = (_E // _NW) // _CH
_GTAIL = (_E // _NW) - _GFULL * _CH
_SFULL = (_E // _NS) // _CH
_STAIL = (_E // _NS) - _SFULL * _CH

@functools.cache
def _sc_mesh():
  return plsc.VectorSubcoreMesh(core_axis_name="c", subcore_axis_name="s",
                                num_cores=_NC, num_subcores=_NS)


def _mm_body(x_ref, w_ref, b_ref, o_ref):
  o_ref[...] = (
      jnp.dot(x_ref[...], w_ref[...], preferred_element_type=jnp.float32)
      + b_ref[...])


def _mm(x, w, b, bm=2000):
  m, k = x.shape
  n = w.shape[1]
  return pl.pallas_call(
      _mm_body,
      grid=(m // bm,),
      in_specs=[
          pl.BlockSpec((bm, k), lambda i: (i, 0)),
          pl.BlockSpec((k, n), lambda i: (0, 0)),
          pl.BlockSpec((1, n), lambda i: (0, 0)),
      ],
      out_specs=pl.BlockSpec((bm, n), lambda i: (i, 0)),
      out_shape=jax.ShapeDtypeStruct((m, n), jnp.float32),
  )(x, w, b.reshape(1, n))


def _fsfd_body(hs_ref, hd_ref, ws_ref, wd_ref, fs_ref, fd_ref):
  fs_ref[...] = jnp.dot(hs_ref[...], ws_ref[...],
                        preferred_element_type=jnp.float32)
  fd_ref[...] = jnp.dot(hd_ref[...], wd_ref[...],
                        preferred_element_type=jnp.float32)


def _fsfd(h_src, h_dst, ws, wd, bm=2000):
  return pl.pallas_call(
      _fsfd_body,
      grid=(_N // bm,),
      in_specs=[
          pl.BlockSpec((bm, _F), lambda i: (i, 0)),
          pl.BlockSpec((bm, _F), lambda i: (i, 0)),
          pl.BlockSpec((_F, _F), lambda i: (0, 0)),
          pl.BlockSpec((_F, _F), lambda i: (0, 0)),
      ],
      out_specs=[
          pl.BlockSpec((bm, _F), lambda i: (i, 0)),
          pl.BlockSpec((bm, _F), lambda i: (i, 0)),
      ],
      out_shape=[
          jax.ShapeDtypeStruct((_N, _F), jnp.float32),
          jax.ShapeDtypeStruct((_N, _F), jnp.float32),
      ],
  )(h_src, h_dst, ws, wd)


def _edge_body(fs_ref, fd_ref, abd_ref, bind_ref, w_ref, x_ref):
  t = fs_ref[...] + fd_ref[...]
  t = jnp.maximum(t, 0.2 * t)
  l = jnp.dot(t, abd_ref[...], preferred_element_type=jnp.float32)
  col = lax.broadcasted_iota(jnp.int32, l.shape, 1)
  ex = jnp.where(col < _NHEADS, jnp.exp(l), 0.0)
  x_ref[...] = ex
  w_ref[...] = fs_ref[...] * jnp.dot(ex, bind_ref[...],
                                     preferred_element_type=jnp.float32)


def _edge(FS, FD, abd, bind, bm=4000):
  return pl.pallas_call(
      _edge_body,
      grid=(_E // bm,),
      in_specs=[
          pl.BlockSpec((bm, _F), lambda i: (i, 0)),
          pl.BlockSpec((bm, _F), lambda i: (i, 0)),
          pl.BlockSpec((_F, _F), lambda i: (0, 0)),
          pl.BlockSpec((_F, _F), lambda i: (0, 0)),
      ],
      out_specs=[
          pl.BlockSpec((bm, _F), lambda i: (i, 0)),
          pl.BlockSpec((bm, _F), lambda i: (i, 0)),
      ],
      out_shape=[
          jax.ShapeDtypeStruct((_E, _F), jnp.float32),
          jax.ShapeDtypeStruct((_E, _F), jnp.float32),
      ],
  )(FS, FD, abd, bind)


def _fin_body(aw_ref, ax_ref, bind_ref, b_ref, o_ref):
  s = jnp.dot(ax_ref[...], bind_ref[...], preferred_element_type=jnp.float32)
  o_ref[...] = aw_ref[...] / (s + 1e-9) + b_ref[...]


def _finalize(aw, ax, bind, b, bm=2000):
  return pl.pallas_call(
      _fin_body,
      grid=(_N // bm,),
      in_specs=[
          pl.BlockSpec((bm, _F), lambda i: (i, 0)),
          pl.BlockSpec((bm, _F), lambda i: (i, 0)),
          pl.BlockSpec((_F, _F), lambda i: (0, 0)),
          pl.BlockSpec((1, _F), lambda i: (0, 0)),
      ],
      out_specs=pl.BlockSpec((bm, _F), lambda i: (i, 0)),
      out_shape=jax.ShapeDtypeStruct((_N, _F), jnp.float32),
  )(aw, ax, bind, b.reshape(1, _F))


def _clf_body(h_ref, w1_ref, b1_ref, w2_ref, b2_ref, o_ref):
  x = jnp.dot(h_ref[...], w1_ref[...], preferred_element_type=jnp.float32)
  x = jnp.maximum(x + b1_ref[...], 0.0)
  o_ref[...] = (
      jnp.dot(x, w2_ref[...], preferred_element_type=jnp.float32)
      + b2_ref[...])


def _clf(h, w1, b1, w2, b2, bm=2000):
  n1 = w1.shape[1]
  n2 = w2.shape[1]
  return pl.pallas_call(
      _clf_body,
      grid=(_N // bm,),
      in_specs=[
          pl.BlockSpec((bm, _F), lambda i: (i, 0)),
          pl.BlockSpec((_F, n1), lambda i: (0, 0)),
          pl.BlockSpec((1, n1), lambda i: (0, 0)),
          pl.BlockSpec((n1, n2), lambda i: (0, 0)),
          pl.BlockSpec((1, n2), lambda i: (0, 0)),
      ],
      out_specs=pl.BlockSpec((bm, n2), lambda i: (i, 0)),
      out_shape=jax.ShapeDtypeStruct((_N, n2), jnp.float32),
  )(h, w1, b1.reshape(1, n1), w2, b2.reshape(1, n2))


def _gather_impl(fs_hbm, fd_hbm, src_hbm, dst_hbm, FS_hbm, FD_hbm,
                 sidx0, didx0, srows0, drows0, sidx1, didx1, srows1, drows1,
                 asem0, asem1, gsem0, gsem1, wsem0, wsem1):
  c = lax.axis_index("c")
  s = lax.axis_index("s")
  wid = s * _NC + c
  base = wid * (_E // _NW)
  sets = ((sidx0, didx0, srows0, drows0, asem0, gsem0, wsem0),
          (sidx1, didx1, srows1, drows1, asem1, gsem1, wsem1))

  def idx_load(j, S):
    si, di = S[0], S[1]
    off = base + j * _CH
    pltpu.async_copy(src_hbm.at[pl.ds(off, _CH)], si, S[4])
    pltpu.async_copy(dst_hbm.at[pl.ds(off, _CH)], di, S[4])

  idx_load(0, sets[0])
  idx_load(1, sets[1])

  def step(j, S):
    si, di, sr, dr, asem, gsem, wsem = S
    off = base + j * _CH
    pltpu.make_async_copy(src_hbm.at[pl.ds(off, _CH)], si, asem).wait()
    pltpu.make_async_copy(dst_hbm.at[pl.ds(off, _CH)], di, asem).wait()

    @pl.when(j >= 2)
    def _rows_free():
      pltpu.make_async_copy(sr, FS_hbm.at[pl.ds(off, _CH)], wsem).wait()
      pltpu.make_async_copy(dr, FD_hbm.at[pl.ds(off, _CH)], wsem).wait()

    cp1 = pltpu.async_copy(fs_hbm.at[si], sr, gsem)
    cp2 = pltpu.async_copy(fd_hbm.at[di], dr, gsem)
    cp1.wait()
    cp2.wait()
    pltpu.async_copy(sr, FS_hbm.at[pl.ds(off, _CH)], wsem)
    pltpu.async_copy(dr, FD_hbm.at[pl.ds(off, _CH)], wsem)

    @pl.when(j + 2 < _GFULL)
    def _next_idx():
      idx_load(j + 2, S)

  def body(j, carry):
    @pl.when(lax.rem(j, 2) == 0)
    def _even():
      step(j, sets[0])

    @pl.when(lax.rem(j, 2) == 1)
    def _odd():
      step(j, sets[1])

    return carry

  lax.fori_loop(0, _GFULL, body, 0, unroll=2)

  for b, S in enumerate(sets):
    off = base + (_GFULL - 2 + b) * _CH
    pltpu.make_async_copy(S[2], FS_hbm.at[pl.ds(off, _CH)], S[6]).wait()
    pltpu.make_async_copy(S[3], FD_hbm.at[pl.ds(off, _CH)], S[6]).wait()

  toff = base + _GFULL * _CH
  si, di, sr, dr, asem, gsem, wsem = sets[0]
  pltpu.sync_copy(src_hbm.at[pl.ds(toff, _GTAIL)], si.at[pl.ds(0, _GTAIL)])
  pltpu.sync_copy(dst_hbm.at[pl.ds(toff, _GTAIL)], di.at[pl.ds(0, _GTAIL)])
  pltpu.async_copy(fs_hbm.at[si.at[pl.ds(0, _GTAIL)]],
                   sr.at[pl.ds(0, _GTAIL)], gsem).wait()
  pltpu.async_copy(fd_hbm.at[di.at[pl.ds(0, _GTAIL)]],
                   dr.at[pl.ds(0, _GTAIL)], gsem).wait()
  pltpu.sync_copy(sr.at[pl.ds(0, _GTAIL)], FS_hbm.at[pl.ds(toff, _GTAIL)])
  pltpu.sync_copy(dr.at[pl.ds(0, _GTAIL)], FD_hbm.at[pl.ds(toff, _GTAIL)])


@functools.cache
def _gather2_kernel():
  return pl.kernel(
      _gather_impl,
      out_type=[
          jax.ShapeDtypeStruct((_E, _F), jnp.float32),
          jax.ShapeDtypeStruct((_E, _F), jnp.float32),
      ],
      mesh=_sc_mesh(),
      scratch_types=[
          pltpu.VMEM((_CH,), jnp.int32),
          pltpu.VMEM((_CH,), jnp.int32),
          pltpu.VMEM((_CH, _F), jnp.float32),
          pltpu.VMEM((_CH, _F), jnp.float32),
          pltpu.VMEM((_CH,), jnp.int32),
          pltpu.VMEM((_CH,), jnp.int32),
          pltpu.VMEM((_CH, _F), jnp.float32),
          pltpu.VMEM((_CH, _F), jnp.float32),
          pltpu.SemaphoreType.DMA,
          pltpu.SemaphoreType.DMA,
          pltpu.SemaphoreType.DMA,
          pltpu.SemaphoreType.DMA,
          pltpu.SemaphoreType.DMA,
          pltpu.SemaphoreType.DMA,
      ],
  )


def _gather2(fs, fd, src, dst):
  return _gather2_kernel()(fs, fd, src, dst)


def _scatter_impl(W_hbm, dst_hbm, z_hbm, out_hbm, acc,
                  idx0, lidx0, rows0, idx1, lidx1, rows1,
                  asem0, ssem0, asem1, ssem1):
  c = lax.axis_index("c")
  s = lax.axis_index("s")
  pltpu.sync_copy(z_hbm.at[pl.ds(s * _ST, _ST)], acc.at[pl.ds(s * _ST, _ST)])

  @pl.when(s == 0)
  def _zero_tail():
    pltpu.sync_copy(z_hbm.at[pl.ds(_NS * _ST, _ZTAIL)],
                    acc.at[pl.ds(_NS * _ST, _ZTAIL)])

  plsc.subcore_barrier()

  lo = (c * _HALF).astype(jnp.int32)
  eps = _E // _NS
  sets = ((idx0, lidx0, rows0, asem0, ssem0),
          (idx1, lidx1, rows1, asem1, ssem1))

  def loads(j, S):
    off = s * eps + j * _CH
    pltpu.async_copy(dst_hbm.at[pl.ds(off, _CH)], S[0], S[3])
    pltpu.async_copy(W_hbm.at[pl.ds(off, _CH)], S[2], S[3])

  loads(0, sets[0])

  def step(j, S, T):
    idx, lidx, rows, asem, ssem = S
    off = s * eps + j * _CH
    pltpu.make_async_copy(dst_hbm.at[pl.ds(off, _CH)], idx, asem).wait()
    pltpu.make_async_copy(W_hbm.at[pl.ds(off, _CH)], rows, asem).wait()
    for l in range(_CH // 16):
      v = idx[pl.ds(l * 16, 16)] - lo
      ok = (v >= 0) & (v < _HALF)
      lidx[pl.ds(l * 16, 16)] = jnp.where(ok, v, _DUMP)
    pltpu.async_copy(rows, acc.at[lidx], ssem, add=True)

    @pl.when(j >= 1)
    def _prev_done():
      pltpu.make_async_copy(T[2], acc.at[T[1]], T[4]).wait()

    @pl.when(j + 1 < _SFULL)
    def _prefetch():
      loads(j + 1, T)

  def body(j, carry):
    @pl.when(lax.rem(j, 2) == 0)
    def _even():
      step(j, sets[0], sets[1])

    @pl.when(lax.rem(j, 2) == 1)
    def _odd():
      step(j, sets[1], sets[0])

    return carry

  lax.fori_loop(0, _SFULL, body, 0, unroll=2)

  pltpu.make_async_copy(sets[1][2], acc.at[sets[1][1]], sets[1][4]).wait()

  toff = s * eps + _SFULL * _CH
  idx, lidx, rows = sets[0][0], sets[0][1], sets[0][2]
  pltpu.sync_copy(dst_hbm.at[pl.ds(toff, _STAIL)], idx.at[pl.ds(0, _STAIL)])
  pltpu.sync_copy(W_hbm.at[pl.ds(toff, _STAIL)], rows.at[pl.ds(0, _STAIL)])
  for l in range(_CH // 16):
    if l < _STAIL // 16:
      v = idx[pl.ds(l * 16, 16)] - lo
      ok = (v >= 0) & (v < _HALF)
      lidx[pl.ds(l * 16, 16)] = jnp.where(ok, v, _DUMP)
    else:
      lidx[pl.ds(l * 16, 16)] = jnp.full((16,), _DUMP, jnp.int32)
  pltpu.sync_copy(rows, acc.at[lidx], add=True)

  plsc.subcore_barrier()

  r = s * _ST
  pltpu.sync_copy(acc.at[pl.ds(r, _ST)],
                  out_hbm.at[pl.ds(c * _HALF + r, _ST)])

  @pl.when(s == 0)
  def _write_tail():
    pltpu.sync_copy(acc.at[pl.ds(_NS * _ST, _WTAIL)],
                    out_hbm.at[pl.ds(c * _HALF + _NS * _ST, _WTAIL)])


@functools.cache
def _scatter_kernel():
  return pl.kernel(
      _scatter_impl,
      out_type=jax.ShapeDtypeStruct((_N, _F), jnp.float32),
      mesh=_sc_mesh(),
      scratch_types=[
          pltpu.VMEM_SHARED((_AROWS, _F), jnp.float32),
          pltpu.VMEM((_CH,), jnp.int32),
          pltpu.VMEM((_CH,), jnp.int32),
          pltpu.VMEM((_CH, _F), jnp.float32),
          pltpu.VMEM((_CH,), jnp.int32),
          pltpu.VMEM((_CH,), jnp.int32),
          pltpu.VMEM((_CH, _F), jnp.float32),
          pltpu.SemaphoreType.DMA,
          pltpu.SemaphoreType.DMA,
          pltpu.SemaphoreType.DMA,
          pltpu.SemaphoreType.DMA,
      ],
  )


def _scatter128(W, dst, z):
  return _scatter_kernel()(W, dst, z)


def _abd(attn):
  a = jnp.zeros((_F, _F), jnp.float32)
  for h in range(_NHEADS):
    a = a.at[h * 32:(h + 1) * 32, h].set(attn[h])
  return a


def _bind():
  b = jnp.zeros((_F, _F), jnp.float32)
  for h in range(_NHEADS):
    b = b.at[h, h * 32:(h + 1) * 32].set(1.0)
  return b


def kernel(h_document, h_word, edge_index_w2d, edge_index_d2w, params):
  pr = params['proj']
  hd0 = _mm(h_document, pr['document']['W'], pr['document']['b'])
  hw0 = _mm(h_word, pr['word']['W'], pr['word']['b'])

  z = jnp.zeros((_AROWS, _F), jnp.float32)
  bind = _bind()

  def gatv2(h_src, h_dst, src, dst, p):
    fs, fd = _fsfd(h_src, h_dst, p['Ws'], p['Wd'])
    FS, FD = _gather2(fs, fd, src, dst)
    Wr, Xr = _edge(FS, FD, _abd(p['attn']), bind)
    aw = _scatter128(Wr, dst, z)
    ax = _scatter128(Xr, dst, z)
    return _finalize(aw, ax, bind, p['b'])

  s_w2d = edge_index_w2d[0].astype(jnp.int32)
  d_w2d = edge_index_w2d[1].astype(jnp.int32)
  s_d2w = edge_index_d2w[0].astype(jnp.int32)
  d_d2w = edge_index_d2w[1].astype(jnp.int32)

  L = params['layers']
  hd1 = gatv2(hw0, hd0, s_w2d, d_w2d, L[0]['w2d'])
  hw1 = gatv2(hd0, hw0, s_d2w, d_d2w, L[0]['d2w'])
  hd2 = gatv2(hw1, hd1, s_w2d, d_w2d, L[1]['w2d'])

  c = params['clf']
  return _clf(hd2, c['W1'], c['b1'], c['W2'], c['b2'])
